# Initial kernel scaffold; baseline (speedup 1.0000x reference)
import jax
import jax.numpy as jnp
from jax.experimental import pallas as pl


def kernel(x, edge_index, W1, b1, ln_gamma, ln_beta, W2, b2):
    raise NotImplementedError("write your pallas kernel here")


# speedup vs baseline: 21.0678x; 21.0678x over previous
import functools

import jax
import jax.numpy as jnp
from jax import lax
from jax.experimental import pallas as pl
from jax.experimental.pallas import tpu as pltpu
from jax.experimental.pallas import tpu_sc as plsc

N = 10000
D = 128
DH = D // 2
E = 320000
E2 = 2 * E
NC = 2
NS = 16
NW = NC * NS
EPW = E2 // NW
B = 80
NB = EPW // B
NZB = N // B

_mesh = plsc.VectorSubcoreMesh(core_axis_name="c", subcore_axis_name="s")


@functools.partial(
    pl.kernel,
    out_type=[jax.ShapeDtypeStruct((N,), jnp.float32),
              jax.ShapeDtypeStruct((N,), jnp.float32)],
    mesh=_mesh,
    scratch_types=[
        pltpu.VMEM((NB, B), jnp.int32),
        pltpu.VMEM((B,), jnp.float32),
        pltpu.VMEM((B,), jnp.float32),
        pltpu.VMEM_SHARED((N,), jnp.float32),
    ],
)
def _deg_kernel(dst_hbm, out0_hbm, out1_hbm, idx_v, ones_v, bnc_v, acc):
    cid = lax.axis_index("c")
    sid = lax.axis_index("s")
    wid = sid * NC + cid
    pltpu.sync_copy(dst_hbm.at[wid], idx_v)
    for c in range(B // 16):
        ones_v[pl.ds(c * 16, 16)] = jnp.ones((16,), jnp.float32)
        bnc_v[pl.ds(c * 16, 16)] = jnp.zeros((16,), jnp.float32)

    @pl.loop(sid, NZB, step=NS)
    def _(b):
        pltpu.sync_copy(bnc_v, acc.at[pl.ds(b * B, B)])

    plsc.subcore_barrier()

    @pl.loop(0, NB)
    def _(j):
        pltpu.sync_copy(ones_v, acc.at[idx_v.at[j]], add=True)

    plsc.subcore_barrier()

    @pl.loop(sid, NZB, step=NS)
    def _(b):
        pltpu.sync_copy(acc.at[pl.ds(b * B, B)], bnc_v)

        @pl.when(cid == 0)
        def _():
            pltpu.sync_copy(bnc_v, out0_hbm.at[pl.ds(b * B, B)])

        @pl.when(cid == 1)
        def _():
            pltpu.sync_copy(bnc_v, out1_hbm.at[pl.ds(b * B, B)])


@functools.partial(
    pl.kernel,
    out_type=[jax.ShapeDtypeStruct((NC, N, DH), jnp.float32),
              jax.ShapeDtypeStruct((NC, N, DH), jnp.float32)],
    mesh=_mesh,
    scratch_types=[
        pltpu.VMEM((NB, B), jnp.int32),
        pltpu.VMEM((NB, B), jnp.int32),
        pltpu.VMEM((B, DH), jnp.float32),
        pltpu.VMEM((B, DH), jnp.float32),
        pltpu.VMEM((B, DH), jnp.float32),
        pltpu.VMEM_SHARED((N, DH), jnp.float32),
        pltpu.SemaphoreType.DMA,
        pltpu.SemaphoreType.DMA,
    ],
    compiler_params=pltpu.CompilerParams(use_tc_tiling_on_sc=False),
)
def _agg_kernel(hlo_hbm, hhi_hbm, src_hbm, dst_hbm, outlo_hbm, outhi_hbm,
                src_v, dst_v, rows0, rows1, zbuf, acc, sem0, sem1):
    cid = lax.axis_index("c")
    sid = lax.axis_index("s")
    wid = sid * NC + cid
    pltpu.sync_copy(src_hbm.at[wid], src_v)
    pltpu.sync_copy(dst_hbm.at[wid], dst_v)

    @pl.loop(0, B)
    def _(r):
        for c in range(DH // 16):
            zbuf[r, pl.ds(c * 16, 16)] = jnp.zeros((16,), jnp.float32)

    @pl.loop(sid, NZB, step=NS)
    def _(b):
        pltpu.sync_copy(zbuf, acc.at[pl.ds(b * B, B)])

    plsc.subcore_barrier()

    for half in range(2):
        h_hbm = hlo_hbm if half == 0 else hhi_hbm
        out_hbm = outlo_hbm if half == 0 else outhi_hbm

        pltpu.async_copy(h_hbm.at[src_v.at[0]], rows0, sem0)

        @pl.loop(0, NB, step=2)
        def _(j):
            pltpu.make_async_copy(h_hbm.at[src_v.at[j]], rows0, sem0).wait()
            pltpu.async_copy(h_hbm.at[src_v.at[j + 1]], rows1, sem1)
            pltpu.sync_copy(rows0, acc.at[dst_v.at[j]], add=True)
            pltpu.make_async_copy(h_hbm.at[src_v.at[j + 1]], rows1, sem1).wait()

            @pl.when(j + 2 < NB)
            def _():
                pltpu.async_copy(h_hbm.at[src_v.at[j + 2]], rows0, sem0)

            pltpu.sync_copy(rows1, acc.at[dst_v.at[j + 1]], add=True)

        plsc.subcore_barrier()

        @pl.loop(sid, NZB, step=NS)
        def _(b):
            pltpu.sync_copy(acc.at[pl.ds(b * B, B)], rows0)
            pltpu.sync_copy(rows0, out_hbm.at[cid, pl.ds(b * B, B)])
            if half == 0:
                pltpu.sync_copy(zbuf, acc.at[pl.ds(b * B, B)])

        plsc.subcore_barrier()


RB = 400


def _dis_body(d0_ref, d1_ref, dis_ref):
    d = d0_ref[...] + d1_ref[...]
    dis_ref[...] = jnp.where(d > 0.0, 1.0 / jnp.sqrt(jnp.maximum(d, 1.0)), 0.0)


def _dis_tc(d0, d1):
    return pl.pallas_call(
        _dis_body,
        grid=(1,),
        in_specs=[pl.BlockSpec((1, N), lambda i: (0, 0)),
                  pl.BlockSpec((1, N), lambda i: (0, 0))],
        out_specs=pl.BlockSpec((1, N), lambda i: (0, 0)),
        out_shape=jax.ShapeDtypeStruct((1, N), jnp.float32),
    )(d0, d1)


def _mm_scale_body(x_ref, w_ref, dis_ref, olo_ref, ohi_ref):
    h = jnp.dot(x_ref[...], w_ref[...], preferred_element_type=jnp.float32)
    h = h * dis_ref[...]
    olo_ref[...] = h[:, :DH]
    ohi_ref[...] = h[:, DH:]


def _mm_scale_tc(x, W, dis):
    return pl.pallas_call(
        _mm_scale_body,
        grid=(N // RB,),
        in_specs=[
            pl.BlockSpec((RB, D), lambda i: (i, 0)),
            pl.BlockSpec((D, D), lambda i: (0, 0)),
            pl.BlockSpec((RB, 1), lambda i: (i, 0)),
        ],
        out_specs=[pl.BlockSpec((RB, DH), lambda i: (i, 0)),
                   pl.BlockSpec((RB, DH), lambda i: (i, 0))],
        out_shape=[jax.ShapeDtypeStruct((N, DH), jnp.float32),
                   jax.ShapeDtypeStruct((N, DH), jnp.float32)],
    )(x, W, dis)


def _mid_body(p0l_ref, p1l_ref, p0h_ref, p1h_ref, dis_ref, b1_ref, g_ref,
              be_ref, w2_ref, olo_ref, ohi_ref):
    dis = dis_ref[...]
    t = jnp.concatenate([p0l_ref[...] + p1l_ref[...],
                         p0h_ref[...] + p1h_ref[...]], axis=1)
    t = t * dis + b1_ref[...]
    mu = jnp.mean(t, axis=1, keepdims=True)
    var = jnp.mean((t - mu) ** 2, axis=1, keepdims=True)
    t = (t - mu) / jnp.sqrt(var + 1e-5) * g_ref[...] + be_ref[...]
    h = jnp.maximum(t, 0.0)
    h = jnp.dot(h, w2_ref[...], preferred_element_type=jnp.float32) * dis
    olo_ref[...] = h[:, :DH]
    ohi_ref[...] = h[:, DH:]


def _mid_tc(p0l, p1l, p0h, p1h, dis, b1, g, be, W2):
    half_spec = pl.BlockSpec((RB, DH), lambda i: (i, 0))
    row_spec = pl.BlockSpec((1, D), lambda i: (0, 0))
    return pl.pallas_call(
        _mid_body,
        grid=(N // RB,),
        in_specs=[
            half_spec, half_spec, half_spec, half_spec,
            pl.BlockSpec((RB, 1), lambda i: (i, 0)),
            row_spec, row_spec, row_spec,
            pl.BlockSpec((D, D), lambda i: (0, 0)),
        ],
        out_specs=[half_spec, half_spec],
        out_shape=[jax.ShapeDtypeStruct((N, DH), jnp.float32),
                   jax.ShapeDtypeStruct((N, DH), jnp.float32)],
    )(p0l, p1l, p0h, p1h, dis, b1, g, be, W2)


def _final_body(q0l_ref, q1l_ref, q0h_ref, q1h_ref, dis_ref, b2_ref, o_ref):
    t = jnp.concatenate([q0l_ref[...] + q1l_ref[...],
                         q0h_ref[...] + q1h_ref[...]], axis=1)
    o_ref[...] = t * dis_ref[...] + b2_ref[...]


def _final_tc(q0l, q1l, q0h, q1h, dis, b2):
    half_spec = pl.BlockSpec((RB, DH), lambda i: (i, 0))
    return pl.pallas_call(
        _final_body,
        grid=(N // RB,),
        in_specs=[
            half_spec, half_spec, half_spec, half_spec,
            pl.BlockSpec((RB, 1), lambda i: (i, 0)),
            pl.BlockSpec((1, D), lambda i: (0, 0)),
        ],
        out_specs=pl.BlockSpec((RB, D), lambda i: (i, 0)),
        out_shape=jax.ShapeDtypeStruct((N, D), jnp.float32),
    )(q0l, q1l, q0h, q1h, dis, b2)


def kernel(x, edge_index, W1, b1, ln_gamma, ln_beta, W2, b2):
    ei = edge_index.astype(jnp.int32)
    src0, dst0 = ei[0], ei[1]
    src_all = jnp.concatenate([src0, dst0]).reshape(NW, NB, B)
    dst_all = jnp.concatenate([dst0, src0]).reshape(NW, NB, B)

    d0, d1 = _deg_kernel(dst_all)
    dis = _dis_tc(d0.reshape(1, N), d1.reshape(1, N)).reshape(N, 1)

    h1l, h1h = _mm_scale_tc(x, W1, dis)
    pl_, ph = _agg_kernel(h1l, h1h, src_all, dst_all)
    h2l, h2h = _mid_tc(pl_[0], pl_[1], ph[0], ph[1], dis, b1.reshape(1, D),
                       ln_gamma.reshape(1, D), ln_beta.reshape(1, D), W2)
    ql, qh = _agg_kernel(h2l, h2h, src_all, dst_all)
    return _final_tc(ql[0], ql[1], qh[0], qh[1], dis, b2.reshape(1, D))
